# Initial kernel scaffold; baseline (speedup 1.0000x reference)
import functools

import jax
import jax.numpy as jnp
from jax import lax
from jax.experimental import pallas as pl
from jax.experimental.pallas import tpu as pltpu
from jax.experimental.pallas import tpu_sc as plsc

N = 10000
E = 320000
D = 128
DH = 64
B = 128
NBLK = E // B
NTILE = 16
NPAD = 10240
ROWS_PER_TILE = NPAD // NTILE

_MESH = plsc.VectorSubcoreMesh(core_axis_name="c", subcore_axis_name="s")
_f32 = jnp.float32


@functools.partial(
    pl.kernel,
    mesh=_MESH,
    out_type=jax.ShapeDtypeStruct((2, NPAD), _f32),
    scratch_types=[
        pltpu.VMEM((1, B), jnp.int32),
        pltpu.VMEM((B,), _f32),
        pltpu.VMEM_SHARED((NPAD,), _f32),
    ],
)
def _sc_degree(dst_hbm, zeros1_hbm, ones_hbm, out_hbm, idx_v, ones_v, acc_sh):
    c = lax.axis_index("c")
    s = lax.axis_index("s")
    w = c * NTILE + s
    pltpu.sync_copy(zeros1_hbm, acc_sh.at[pl.ds(s * ROWS_PER_TILE, ROWS_PER_TILE)])
    pltpu.sync_copy(ones_hbm, ones_v)
    plsc.subcore_barrier()

    @pl.loop(w, NBLK, step=2 * NTILE)
    def _(g):
        pltpu.sync_copy(dst_hbm.at[pl.ds(g * B, B)], idx_v.at[0])
        pltpu.sync_copy(ones_v, acc_sh.at[idx_v.at[0]], add=True)

    plsc.subcore_barrier()
    pltpu.sync_copy(
        acc_sh.at[pl.ds(s * ROWS_PER_TILE, ROWS_PER_TILE)],
        out_hbm.at[c, pl.ds(s * ROWS_PER_TILE, ROWS_PER_TILE)],
    )


@functools.partial(
    pl.kernel,
    mesh=_MESH,
    out_type=[
        jax.ShapeDtypeStruct((NPAD, DH), _f32),
        jax.ShapeDtypeStruct((NPAD, DH), _f32),
    ],
    scratch_types=[
        pltpu.VMEM((1, B), jnp.int32),
        pltpu.VMEM((1, B), jnp.int32),
        pltpu.VMEM((B, DH), _f32),
        pltpu.VMEM_SHARED((NPAD, DH), _f32),
        pltpu.SemaphoreType.DMA,
    ],
)
def _sc_prop(tlo_hbm, thi_hbm, src_hbm, dst_hbm, zeros2_hbm,
             olo_hbm, ohi_hbm, src_v, dst_v, rows_v, acc_sh, sem):
    c = lax.axis_index("c")
    s = lax.axis_index("s")
    pltpu.sync_copy(zeros2_hbm, acc_sh.at[pl.ds(s * ROWS_PER_TILE, ROWS_PER_TILE), :])
    plsc.subcore_barrier()

    def edge_block(g, t_hbm):
        pltpu.sync_copy(src_hbm.at[pl.ds(g * B, B)], src_v.at[0])
        pltpu.sync_copy(dst_hbm.at[pl.ds(g * B, B)], dst_v.at[0])
        pltpu.async_copy(t_hbm.at[src_v.at[0]], rows_v, sem).wait()
        pltpu.sync_copy(rows_v, acc_sh.at[dst_v.at[0]], add=True)

    @pl.when(c == 0)
    def _():
        @pl.loop(s, NBLK, step=NTILE)
        def _(g):
            edge_block(g, tlo_hbm)

    @pl.when(c == 1)
    def _():
        @pl.loop(s, NBLK, step=NTILE)
        def _(g):
            edge_block(g, thi_hbm)

    plsc.subcore_barrier()
    tile_rows = pl.ds(s * ROWS_PER_TILE, ROWS_PER_TILE)

    @pl.when(c == 0)
    def _():
        pltpu.sync_copy(acc_sh.at[tile_rows, :], olo_hbm.at[tile_rows, :])

    @pl.when(c == 1)
    def _():
        pltpu.sync_copy(acc_sh.at[tile_rows, :], ohi_hbm.at[tile_rows, :])


def _stage_scale0(dp0_ref, dp1_ref, feat_ref,
                  t0lo_ref, t0hi_ref, dinv_ref, dinv2_ref):
    deg = dp0_ref[...] + dp1_ref[...] + 1.0
    di = lax.rsqrt(deg)
    dinv_ref[...] = di
    dinv2_ref[...] = 1.0 / deg
    t0 = feat_ref[...] * di
    t0lo_ref[...] = t0[:, :DH]
    t0hi_ref[...] = t0[:, DH:]


def _stage_mid(ulo_ref, uhi_ref, t0lo_ref, t0hi_ref, dinv2_ref,
               t1lo_ref, t1hi_ref):
    di2 = dinv2_ref[...]
    t1lo_ref[...] = (ulo_ref[...] + t0lo_ref[...]) * di2
    t1hi_ref[...] = (uhi_ref[...] + t0hi_ref[...]) * di2


def _stage_final(ulo_ref, uhi_ref, t1lo_ref, t1hi_ref, dinv_ref,
                 w_ref, b_ref, out_ref):
    di = dinv_ref[...]
    h_lo = (ulo_ref[...] + t1lo_ref[...]) * di
    h_hi = (uhi_ref[...] + t1hi_ref[...]) * di
    h = jnp.concatenate([h_lo, h_hi], axis=1)
    out_ref[...] = (
        jnp.dot(h, w_ref[...], preferred_element_type=jnp.float32) + b_ref[...]
    )


@jax.jit
def kernel(feat, edge_index, W, b):
    src = edge_index[0].astype(jnp.int32)
    dst = edge_index[1].astype(jnp.int32)
    zeros1 = jnp.zeros((ROWS_PER_TILE,), _f32)
    zeros2 = jnp.zeros((ROWS_PER_TILE, DH), _f32)
    ones = jnp.ones((B,), _f32)

    deg_p = _sc_degree(dst, zeros1, ones)
    dp0 = deg_p[0, :N].reshape(N, 1)
    dp1 = deg_p[1, :N].reshape(N, 1)

    t0lo, t0hi, dinv, dinv2 = pl.pallas_call(
        _stage_scale0,
        out_shape=[
            jax.ShapeDtypeStruct((N, DH), _f32),
            jax.ShapeDtypeStruct((N, DH), _f32),
            jax.ShapeDtypeStruct((N, 1), _f32),
            jax.ShapeDtypeStruct((N, 1), _f32),
        ],
    )(dp0, dp1, feat)

    u1lo, u1hi = _sc_prop(t0lo, t0hi, src, dst, zeros2)
    t1lo, t1hi = pl.pallas_call(
        _stage_mid,
        out_shape=[
            jax.ShapeDtypeStruct((N, DH), _f32),
            jax.ShapeDtypeStruct((N, DH), _f32),
        ],
    )(u1lo[:N], u1hi[:N], t0lo, t0hi, dinv2)

    u2lo, u2hi = _sc_prop(t1lo, t1hi, src, dst, zeros2)
    out = pl.pallas_call(
        _stage_final,
        out_shape=jax.ShapeDtypeStruct((N, D), _f32),
    )(u2lo[:N], u2hi[:N], t1lo, t1hi, dinv, W, b)
    return out


# speedup vs baseline: 15.8293x; 15.8293x over previous
import functools

import jax
import jax.numpy as jnp
from jax import lax
from jax.experimental import pallas as pl
from jax.experimental.pallas import tpu as pltpu
from jax.experimental.pallas import tpu_sc as plsc

N = 10000
E = 320000
D = 128
B = 128
NBLK = E // B
NTILE = 16
NW = 32
NPAD = 10240
RPT = NPAD // NTILE

_MESH = plsc.VectorSubcoreMesh(core_axis_name="c", subcore_axis_name="s")
_f32 = jnp.float32


@functools.partial(
    pl.kernel,
    mesh=_MESH,
    out_type=jax.ShapeDtypeStruct((2, NPAD), _f32),
    scratch_types=[
        pltpu.VMEM((1, B), jnp.int32),
        pltpu.VMEM((B,), _f32),
        pltpu.VMEM_SHARED((NPAD,), _f32),
    ],
)
def _sc_degree(dst_hbm, zeros1_hbm, ones_hbm, out_hbm, idx_v, ones_v, acc_sh):
    c = lax.axis_index("c")
    s = lax.axis_index("s")
    w = c * NTILE + s
    pltpu.sync_copy(zeros1_hbm, acc_sh.at[pl.ds(s * RPT, RPT)])
    pltpu.sync_copy(ones_hbm, ones_v)
    plsc.subcore_barrier()

    @pl.loop(w, NBLK, step=NW)
    def _(g):
        pltpu.sync_copy(dst_hbm.at[pl.ds(g * B, B)], idx_v.at[0])
        pltpu.sync_copy(ones_v, acc_sh.at[idx_v.at[0]], add=True)

    plsc.subcore_barrier()
    pltpu.sync_copy(
        acc_sh.at[pl.ds(s * RPT, RPT)],
        out_hbm.at[c, pl.ds(s * RPT, RPT)],
    )


@functools.partial(
    pl.kernel,
    mesh=_MESH,
    out_type=[
        jax.ShapeDtypeStruct((NPAD, D), _f32),
        jax.ShapeDtypeStruct((NPAD, D), _f32),
    ],
    scratch_types=[
        pltpu.VMEM((1, B), jnp.int32),
        pltpu.VMEM((1, B), jnp.int32),
        pltpu.VMEM((B, D), _f32),
        pltpu.VMEM_SHARED((NPAD, D), _f32),
        pltpu.SemaphoreType.DMA,
    ],
)
def _sc_prop(t_hbm, src_hbm, dst_hbm, zeros2_hbm,
             o0_hbm, o1_hbm, src_v, dst_v, rows_v, acc_sh, sem):
    c = lax.axis_index("c")
    s = lax.axis_index("s")
    w = c * NTILE + s
    pltpu.sync_copy(zeros2_hbm, acc_sh.at[pl.ds(s * RPT, RPT), :])
    plsc.subcore_barrier()

    @pl.loop(w, NBLK, step=NW)
    def _(g):
        pltpu.sync_copy(src_hbm.at[pl.ds(g * B, B)], src_v.at[0])
        pltpu.sync_copy(dst_hbm.at[pl.ds(g * B, B)], dst_v.at[0])
        pltpu.async_copy(t_hbm.at[src_v.at[0]], rows_v, sem).wait()
        pltpu.sync_copy(rows_v, acc_sh.at[dst_v.at[0]], add=True)

    plsc.subcore_barrier()
    tile_rows = pl.ds(s * RPT, RPT)

    @pl.when(c == 0)
    def _():
        pltpu.sync_copy(acc_sh.at[tile_rows, :], o0_hbm.at[tile_rows, :])

    @pl.when(c == 1)
    def _():
        pltpu.sync_copy(acc_sh.at[tile_rows, :], o1_hbm.at[tile_rows, :])


def _stage_scale0(dp0_ref, dp1_ref, feat_ref, t0_ref, dinv_ref, dinv2_ref):
    deg = dp0_ref[...] + dp1_ref[...] + 1.0
    di = lax.rsqrt(deg)
    dinv_ref[...] = di
    dinv2_ref[...] = 1.0 / deg
    t0_ref[...] = feat_ref[...] * di


def _stage_mid(u0_ref, u1_ref, t0_ref, dinv2_ref, t1_ref):
    t1_ref[...] = (u0_ref[...] + u1_ref[...] + t0_ref[...]) * dinv2_ref[...]


def _stage_final(u0_ref, u1_ref, t1_ref, dinv_ref, w_ref, b_ref, out_ref):
    h = (u0_ref[...] + u1_ref[...] + t1_ref[...]) * dinv_ref[...]
    out_ref[...] = (
        jnp.dot(h, w_ref[...], preferred_element_type=jnp.float32) + b_ref[...]
    )


@jax.jit
def kernel(feat, edge_index, W, b):
    src = edge_index[0].astype(jnp.int32)
    dst = edge_index[1].astype(jnp.int32)
    zeros1 = jnp.zeros((RPT,), _f32)
    zeros2 = jnp.zeros((RPT, D), _f32)
    ones = jnp.ones((B,), _f32)

    deg_p = _sc_degree(dst, zeros1, ones)
    dp0 = deg_p[0, :N].reshape(N, 1)
    dp1 = deg_p[1, :N].reshape(N, 1)

    t0, dinv, dinv2 = pl.pallas_call(
        _stage_scale0,
        out_shape=[
            jax.ShapeDtypeStruct((N, D), _f32),
            jax.ShapeDtypeStruct((N, 1), _f32),
            jax.ShapeDtypeStruct((N, 1), _f32),
        ],
    )(dp0, dp1, feat)

    u10, u11 = _sc_prop(t0, src, dst, zeros2)
    t1 = pl.pallas_call(
        _stage_mid,
        out_shape=jax.ShapeDtypeStruct((N, D), _f32),
    )(u10[:N], u11[:N], t0, dinv2)

    u20, u21 = _sc_prop(t1, src, dst, zeros2)
    out = pl.pallas_call(
        _stage_final,
        out_shape=jax.ShapeDtypeStruct((N, D), _f32),
    )(u20[:N], u21[:N], t1, dinv, W, b)
    return out
